# baseline (speedup 1.0000x reference)
import functools

import jax
import jax.numpy as jnp
from jax import lax
from jax.experimental import pallas as pl
from jax.experimental.pallas import tpu as pltpu
from jax.experimental.pallas import tpu_sc as plsc

_B, _N, _C, _K = 8, 2048, 7, 16
_R = _B * _N
_BLK = 32
_HI = lax.Precision.HIGHEST
_F32 = jnp.float32

_NC, _NS, _L = 2, 16, 16
_NW = _NC * _NS


def _knn_mlp1_body(pts_ref, w1a, b1a, w1b, b1b, w1c, b1c, x1_ref, idx_ref,
                   s_ref):
    b = pl.program_id(0)
    i = pl.program_id(1)
    p_all = pts_ref[0]
    p_blk = pts_ref[0, pl.ds(i * _BLK, _BLK), :]

    xx_all = jnp.sum(p_all * p_all, axis=1)
    xx_blk = jnp.sum(p_blk * p_blk, axis=1)
    dotp = lax.dot_general(p_blk, p_all, (((1,), (1,)), ((), ())),
                           precision=_HI, preferred_element_type=_F32)
    inner = -2.0 * dotp
    s = (-xx_all)[None, :] - inner
    s_ref[...] = s - xx_blk[:, None]

    iota = lax.broadcasted_iota(jnp.int32, (_BLK, _N), 1)
    kiota = lax.broadcasted_iota(jnp.int32, (_BLK, _K), 1)
    neg_inf = jnp.float32(-jnp.inf)

    def step(k, picks_acc):
        s = s_ref[...]
        m = jnp.max(s, axis=1, keepdims=True)
        key = jnp.where(s == m, iota, _N)
        a = jnp.min(key, axis=1, keepdims=True)
        s_ref[...] = jnp.where(iota == a, neg_inf, s)
        return jnp.where(kiota == k, a, picks_acc)

    picks = lax.fori_loop(0, _K, step,
                          jnp.zeros((_BLK, _K), jnp.int32))
    idx_ref[0] = picks + b * _N

    oh0 = (iota == picks[:, 0:1]).astype(_F32)
    oh1 = (iota == picks[:, 1:2]).astype(_F32)
    g0 = lax.dot_general(oh0, p_all, (((1,), (0,)), ((), ())),
                         precision=_HI, preferred_element_type=_F32)
    g1 = lax.dot_general(oh1, p_all, (((1,), (0,)), ((), ())),
                         precision=_HI, preferred_element_type=_F32)
    parts = [p_blk] + [g0[:, a:a + 1] * g1 for a in range(_C)]
    f56 = jnp.concatenate(parts, axis=1)

    def dense(x, w, bb):
        return lax.dot_general(x, w[...], (((1,), (1,)), ((), ())),
                               precision=_HI, preferred_element_type=_F32) + bb[...]

    h = jnp.maximum(dense(f56, w1a, b1a), 0.0)
    h = jnp.maximum(dense(h, w1b, b1b), 0.0)
    h = jnp.maximum(dense(h, w1c, b1c), 0.0)
    x1_ref[0] = jnp.concatenate([h, jnp.zeros((_BLK, 64), _F32)], axis=1)


def _knn_mlp1(pts, W1a, b1a, W1b, b1b, W1c, b1c):
    grid = (_B, _N // _BLK)
    full = lambda shape: pl.BlockSpec(shape, lambda b, i: tuple(0 for _ in shape))
    return pl.pallas_call(
        _knn_mlp1_body,
        grid=grid,
        in_specs=[
            pl.BlockSpec((1, _N, _C), lambda b, i: (b, 0, 0)),
            full((64, 56)), full((1, 64)),
            full((64, 64)), full((1, 64)),
            full((64, 64)), full((1, 64)),
        ],
        out_specs=[
            pl.BlockSpec((1, _BLK, 128), lambda b, i: (b, i, 0)),
            pl.BlockSpec((1, _BLK, _K), lambda b, i: (b, i, 0)),
        ],
        out_shape=[
            jax.ShapeDtypeStruct((_B, _N, 128), _F32),
            jax.ShapeDtypeStruct((_B, _N, _K), jnp.int32),
        ],
        scratch_shapes=[pltpu.VMEM((_BLK, _N), _F32)],
    )(pts, W1a, b1a.reshape(1, -1), W1b, b1b.reshape(1, -1),
      W1c, b1c.reshape(1, -1))


def _sc_maxpool(x, idx2d, d):
    rpw = _R // _NW
    nch = rpw // 8
    mesh = plsc.VectorSubcoreMesh(core_axis_name="c", subcore_axis_name="s")

    @functools.partial(
        pl.kernel,
        out_type=jax.ShapeDtypeStruct((_R, 128), _F32),
        mesh=mesh,
        scratch_types=[
            pltpu.VMEM((nch, 128), jnp.int32),
            pltpu.VMEM((128, 128), _F32),
            pltpu.VMEM((128, 128), _F32),
            pltpu.VMEM((rpw, 128), _F32),
            pltpu.SemaphoreType.DMA,
            pltpu.SemaphoreType.DMA,
        ],
    )
    def mp(x_hbm, idx_hbm, out_hbm, idx_v, rows0, rows1, out_v, sem0, sem1):
        wid = lax.axis_index("s") * _NC + lax.axis_index("c")
        base = wid * rpw
        pltpu.sync_copy(idx_hbm.at[pl.ds(wid * nch, nch)], idx_v)
        rows = (rows0, rows1)
        sems = (sem0, sem1)

        def fire(ci, bb):
            pltpu.async_copy(x_hbm.at[idx_v.at[ci]], rows[bb], sems[bb])

        fire(0, 0)
        fire(1, 1)

        def chunk(g, carry):
            for bb in range(2):
                ci = g * 2 + bb
                pltpu.make_async_copy(x_hbm.at[idx_v.at[ci]], rows[bb],
                                      sems[bb]).wait()
                rv = rows[bb]

                def row(r, c2):
                    for c in range(d // _L):
                        acc = rv[r * _K, pl.ds(c * _L, _L)]
                        for nn in range(1, _K):
                            acc = jnp.maximum(
                                acc, rv[r * _K + nn, pl.ds(c * _L, _L)])
                        out_v[ci * 8 + r, pl.ds(c * _L, _L)] = acc
                    return c2

                lax.fori_loop(0, 8, row, 0)

                @pl.when(ci + 2 < nch)
                def _():
                    fire(ci + 2, bb)
            return carry

        lax.fori_loop(0, nch // 2, chunk, 0)
        pltpu.sync_copy(out_v, out_hbm.at[pl.ds(base, rpw)])

    return mp(x, idx2d)


def _mlp2_body(x_ref, wl, bl, wc, bc, o_ref):
    h = lax.dot_general(x_ref[:, :64], wl[...], (((1,), (1,)), ((), ())),
                        precision=_HI, preferred_element_type=_F32) + bl[...]
    h = lax.dot_general(h, wc[...], (((1,), (1,)), ((), ())),
                        precision=_HI, preferred_element_type=_F32) + bc[...]
    o_ref[...] = jnp.maximum(h, 0.0)


def _mlp3_body(x_ref, wl, bl, wc, bc, o_ref):
    h = lax.dot_general(x_ref[...], wl[...], (((1,), (1,)), ((), ())),
                        precision=_HI, preferred_element_type=_F32) + bl[...]
    o_ref[...] = lax.dot_general(h, wc[...], (((1,), (1,)), ((), ())),
                                 precision=_HI, preferred_element_type=_F32) + bc[...]


def _mlp_call(body, x, wl, bl, wc, bc, dmid, dout, blk=2048):
    din = x.shape[1]
    full = lambda shape: pl.BlockSpec(shape, lambda i: tuple(0 for _ in shape))
    return pl.pallas_call(
        body,
        grid=(_R // blk,),
        in_specs=[
            pl.BlockSpec((blk, din), lambda i: (i, 0)),
            full(wl.shape), full((1, dmid)),
            full(wc.shape), full((1, dout)),
        ],
        out_specs=pl.BlockSpec((blk, dout), lambda i: (i, 0)),
        out_shape=jax.ShapeDtypeStruct((_R, dout), _F32),
    )(x, wl, bl.reshape(1, -1), wc, bc.reshape(1, -1))


def kernel(pts, W1a, b1a, W1b, b1b, W1c, b1c, Wl1, bl1, Wc1, bc1,
           Wl2, bl2, Wc2, bc2):
    x1, idx = _knn_mlp1(pts, W1a, b1a, W1b, b1b, W1c, b1c)
    idx2d = idx.reshape(-1, 128)
    m1 = _sc_maxpool(x1.reshape(_R, 128), idx2d, 64)
    x2 = _mlp_call(_mlp2_body, m1, Wl1, bl1, Wc1, bc1, 64, 128)
    m2 = _sc_maxpool(x2, idx2d, 128)
    out = _mlp_call(_mlp3_body, m2, Wl2, bl2, Wc2, bc2, 128, 512)
    return out.reshape(_B, _N, 512)
